# Initial kernel scaffold; baseline (speedup 1.0000x reference)
import functools

import jax
import jax.numpy as jnp
from jax import lax
from jax.experimental import pallas as pl
from jax.experimental.pallas import tpu as pltpu
from jax.experimental.pallas import tpu_sc as plsc

N = 10000
D_IN = 256
D_HID = 256
D_OUT = 128
E = 160000

NTILES = 16
CH = 128
NPAD = 10240
EPAD = 163840
ROWS_PER_TILE = NPAD // NTILES
CHUNKS_PER_TILE = EPAD // (NTILES * CH)
MBLK = 1000
GRID_M = N // MBLK

_MESH = plsc.VectorSubcoreMesh(core_axis_name="c", subcore_axis_name="s")


def _fill(ref, nrows, ncols, value):
    npc = ncols // 16
    v = jnp.full((16,), value, jnp.float32)

    def body(i, _):
        ref[i // npc, pl.ds((i % npc) * 16, 16)] = v
        return 0

    lax.fori_loop(0, nrows * npc, body, 0)


@functools.partial(
    pl.kernel,
    out_type=jax.ShapeDtypeStruct((2, NPAD, 16), jnp.float32),
    mesh=_MESH,
    scratch_types=[
        pltpu.VMEM((CH,), jnp.int32),
        pltpu.VMEM((CH, 16), jnp.float32),
        pltpu.VMEM((CH, 16), jnp.float32),
        pltpu.VMEM_SHARED((NPAD, 16), jnp.float32),
    ],
)
def _deg_kernel(dst_hbm, out_hbm, dst_v, ones_v, zero_v, acc_sh):
    c = lax.axis_index("c")
    s = lax.axis_index("s")
    _fill(ones_v, CH, 16, 1.0)
    _fill(zero_v, CH, 16, 0.0)
    base_row = s * ROWS_PER_TILE
    for z in range(ROWS_PER_TILE // CH):
        pltpu.sync_copy(zero_v, acc_sh.at[pl.ds(base_row + z * CH, CH)])
    plsc.subcore_barrier()

    half = CHUNKS_PER_TILE // 2

    def body(j, _):
        pltpu.sync_copy(dst_hbm.at[s, c * half + j], dst_v)
        pltpu.sync_copy(ones_v, acc_sh.at[dst_v], add=True)
        return 0

    lax.fori_loop(0, half, body, 0)
    plsc.subcore_barrier()
    pltpu.sync_copy(acc_sh.at[pl.ds(base_row, ROWS_PER_TILE)],
                    out_hbm.at[c, pl.ds(base_row, ROWS_PER_TILE)])


def _make_agg(dh):
    @functools.partial(
        pl.kernel,
        out_type=jax.ShapeDtypeStruct((2, NPAD, dh), jnp.float32),
        mesh=_MESH,
        scratch_types=[
            pltpu.VMEM((CH,), jnp.int32),
            pltpu.VMEM((CH,), jnp.int32),
            pltpu.VMEM((CH, dh), jnp.float32),
            pltpu.VMEM_SHARED((NPAD, dh), jnp.float32),
            pltpu.SemaphoreType.DMA,
        ],
    )
    def agg(gtab_hbm, src_hbm, dst_hbm, out_hbm, src_v, dst_v, rows_v,
            acc_sh, sem):
        c = lax.axis_index("c")
        s = lax.axis_index("s")
        _fill(rows_v, CH, dh, 0.0)
        base_row = s * ROWS_PER_TILE
        for z in range(ROWS_PER_TILE // CH):
            pltpu.sync_copy(rows_v, acc_sh.at[pl.ds(base_row + z * CH, CH)])
        plsc.subcore_barrier()

        def body(j, _):
            pltpu.sync_copy(src_hbm.at[c, s, j], src_v)
            pltpu.sync_copy(dst_hbm.at[s, j], dst_v)
            pltpu.async_copy(gtab_hbm.at[src_v], rows_v, sem).wait()
            pltpu.sync_copy(rows_v, acc_sh.at[dst_v], add=True)
            return 0

        lax.fori_loop(0, CHUNKS_PER_TILE, body, 0)
        plsc.subcore_barrier()
        pltpu.sync_copy(acc_sh.at[pl.ds(base_row, ROWS_PER_TILE)],
                        out_hbm.at[c, pl.ds(base_row, ROWS_PER_TILE)])

    return agg


_agg128 = _make_agg(D_HID // 2)
_agg64 = _make_agg(D_OUT // 2)


def _dinv_block(deg_ref):
    d = deg_ref[0, :, 0:1] + deg_ref[1, :, 0:1] + 1.0
    return lax.rsqrt(d)


def _tc1_body(deg_ref, x_ref, w0_ref, g_ref):
    dinv = _dinv_block(deg_ref)
    h = jnp.dot(x_ref[...], w0_ref[...], preferred_element_type=jnp.float32)
    g = h * dinv
    g_ref[0, :, :] = g[:, : D_HID // 2]
    g_ref[1, :, :] = g[:, D_HID // 2:]


def _tc2_body(deg_ref, s0_ref, g0_ref, b0_ref, w1_ref, g1_ref):
    dinv = _dinv_block(deg_ref)
    ya = (s0_ref[0, :, :] + g0_ref[0, :, :]) * dinv
    yb = (s0_ref[1, :, :] + g0_ref[1, :, :]) * dinv
    y = jnp.concatenate([ya, yb], axis=1) + b0_ref[...]
    out0 = jnp.maximum(y, 0.0)
    h1 = jnp.dot(out0, w1_ref[...], preferred_element_type=jnp.float32)
    g1 = h1 * dinv
    g1_ref[0, :, :] = g1[:, : D_OUT // 2]
    g1_ref[1, :, :] = g1[:, D_OUT // 2:]


def _tc3_body(deg_ref, s1_ref, g1_ref, b1_ref, z_ref):
    dinv = _dinv_block(deg_ref)
    za = (s1_ref[0, :, :] + g1_ref[0, :, :]) * dinv
    zb = (s1_ref[1, :, :] + g1_ref[1, :, :]) * dinv
    z = jnp.concatenate([za, zb], axis=1) + b1_ref[...]
    z_ref[...] = jnp.maximum(z, 0.0)


def _deg_spec():
    return pl.BlockSpec((2, MBLK, 16), lambda i: (0, i, 0))


_tc1 = pl.pallas_call(
    _tc1_body,
    grid=(GRID_M,),
    in_specs=[
        _deg_spec(),
        pl.BlockSpec((MBLK, D_IN), lambda i: (i, 0)),
        pl.BlockSpec((D_IN, D_HID), lambda i: (0, 0)),
    ],
    out_specs=pl.BlockSpec((2, MBLK, D_HID // 2), lambda i: (0, i, 0)),
    out_shape=jax.ShapeDtypeStruct((2, N, D_HID // 2), jnp.float32),
)

_tc2 = pl.pallas_call(
    _tc2_body,
    grid=(GRID_M,),
    in_specs=[
        _deg_spec(),
        pl.BlockSpec((2, MBLK, D_HID // 2), lambda i: (0, i, 0)),
        pl.BlockSpec((2, MBLK, D_HID // 2), lambda i: (0, i, 0)),
        pl.BlockSpec((1, D_HID), lambda i: (0, 0)),
        pl.BlockSpec((D_HID, D_OUT), lambda i: (0, 0)),
    ],
    out_specs=pl.BlockSpec((2, MBLK, D_OUT // 2), lambda i: (0, i, 0)),
    out_shape=jax.ShapeDtypeStruct((2, N, D_OUT // 2), jnp.float32),
)

_tc3 = pl.pallas_call(
    _tc3_body,
    grid=(GRID_M,),
    in_specs=[
        _deg_spec(),
        pl.BlockSpec((2, MBLK, D_OUT // 2), lambda i: (0, i, 0)),
        pl.BlockSpec((2, MBLK, D_OUT // 2), lambda i: (0, i, 0)),
        pl.BlockSpec((1, D_OUT), lambda i: (0, 0)),
    ],
    out_specs=pl.BlockSpec((MBLK, D_OUT), lambda i: (i, 0)),
    out_shape=jax.ShapeDtypeStruct((N, D_OUT), jnp.float32),
)


def kernel(x, edge_index, W0, b0, W1, b1):
    src = edge_index[0].astype(jnp.int32)
    dst = edge_index[1].astype(jnp.int32)
    npad = EPAD - E
    src_p = jnp.concatenate([src, jnp.zeros((npad,), jnp.int32)])
    dst_p = jnp.concatenate([dst, jnp.full((npad,), N, jnp.int32)])
    src2 = jnp.stack([src_p, src_p + N]).reshape(2, NTILES, CHUNKS_PER_TILE, CH)
    dst3 = dst_p.reshape(NTILES, CHUNKS_PER_TILE, CH)

    degp = _deg_kernel(dst3)

    g0 = _tc1(degp, x, W0)
    s0 = _agg128(g0.reshape(2 * N, D_HID // 2), src2, dst3)

    g1 = _tc2(degp, s0[:, :N, :], g0, b0.reshape(1, D_HID), W1)
    s1 = _agg64(g1.reshape(2 * N, D_OUT // 2), src2, dst3)

    z = _tc3(degp, s1[:, :N, :], g1, b1.reshape(1, D_OUT))
    return z


# speedup vs baseline: 15.4246x; 15.4246x over previous
import functools

import jax
import jax.numpy as jnp
from jax import lax
from jax.experimental import pallas as pl
from jax.experimental.pallas import tpu as pltpu
from jax.experimental.pallas import tpu_sc as plsc

N = 10000
D_IN = 256
D_HID = 256
D_OUT = 128
E = 160000

NTILES = 16
CH = 128
NPAD = 10240
EPAD = 163840
ROWS_PER_TILE = NPAD // NTILES
CHUNKS_PER_TILE = EPAD // (NTILES * CH)
MBLK = 1000
GRID_M = N // MBLK

_MESH = plsc.VectorSubcoreMesh(core_axis_name="c", subcore_axis_name="s",
                               num_cores=2, num_subcores=NTILES)


def _fill(ref, nrows, ncols, value):
    npc = ncols // 16
    v = jnp.full((16,), value, jnp.float32)

    def body(i, _):
        ref[i // npc, pl.ds((i % npc) * 16, 16)] = v
        return 0

    lax.fori_loop(0, nrows * npc, body, 0)


_HCH = CHUNKS_PER_TILE // 2


@functools.partial(
    pl.kernel,
    out_type=jax.ShapeDtypeStruct((2, NPAD, 128), jnp.float32),
    mesh=_MESH,
    scratch_types=[
        pltpu.VMEM((CH,), jnp.int32),
        pltpu.VMEM((CH, 128), jnp.float32),
        pltpu.VMEM((CH, 128), jnp.float32),
        pltpu.VMEM_SHARED((NPAD, 128), jnp.float32),
    ],
)
def _deg_kernel(dst_hbm, out_hbm, dst_v, ones_v, zero_v, acc_sh):
    c = lax.axis_index("c")
    s = lax.axis_index("s")
    _fill(ones_v, CH, 128, 1.0)
    _fill(zero_v, CH, 128, 0.0)
    base_row = s * ROWS_PER_TILE
    for z in range(ROWS_PER_TILE // CH):
        pltpu.sync_copy(zero_v, acc_sh.at[pl.ds(base_row + z * CH, CH)])
    plsc.subcore_barrier()

    def body(j, _):
        pltpu.sync_copy(dst_hbm.at[c, s, j], dst_v)
        pltpu.sync_copy(ones_v, acc_sh.at[dst_v], add=True)
        return 0

    lax.fori_loop(0, _HCH, body, 0)
    plsc.subcore_barrier()
    pltpu.sync_copy(acc_sh.at[pl.ds(base_row, ROWS_PER_TILE)],
                    out_hbm.at[c, pl.ds(base_row, ROWS_PER_TILE)])


NBUF = 2
RING_D = 4


def _make_agg(dh, nch):
    @functools.partial(
        pl.kernel,
        out_type=jax.ShapeDtypeStruct((2, NPAD, dh), jnp.float32),
        mesh=_MESH,
        scratch_types=[
            pltpu.VMEM((nch, CH), jnp.int32),
            pltpu.VMEM((RING_D, CH), jnp.int32),
            pltpu.VMEM((NBUF, CH, dh), jnp.float32),
            pltpu.VMEM_SHARED((NPAD, dh), jnp.float32),
        ] + [pltpu.SemaphoreType.DMA] * (2 * NBUF + RING_D),
    )
    def agg(gtab_hbm, src_hbm, dst_hbm, out_hbm, src_v, dst_v, rows_v,
            acc_sh, *sems):
        gsem = sems[:NBUF]
        ssem = sems[NBUF:2 * NBUF]
        dsem = sems[2 * NBUF:]
        c = lax.axis_index("c")
        s = lax.axis_index("s")
        _fill(rows_v.at[0], CH, dh, 0.0)
        base_row = s * ROWS_PER_TILE
        for z in range(ROWS_PER_TILE // CH):
            pltpu.sync_copy(rows_v.at[0],
                            acc_sh.at[pl.ds(base_row + z * CH, CH)])
        pltpu.sync_copy(src_hbm.at[c, s], src_v)
        for q in range(2):
            pltpu.async_copy(dst_hbm.at[c, s, q], dst_v.at[q], dsem[q])
        pltpu.async_copy(gtab_hbm.at[src_v.at[0]], rows_v.at[0], gsem[0])
        plsc.subcore_barrier()

        def step(t, _):
            for u in range(4):
                j = t * 4 + u
                b = u % NBUF
                q = u % RING_D
                pltpu.make_async_copy(gtab_hbm.at[src_v.at[j]], rows_v.at[b],
                                      gsem[b]).wait()
                pltpu.make_async_copy(dst_hbm.at[c, s, j], dst_v.at[q],
                                      dsem[q]).wait()
                pltpu.async_copy(rows_v.at[b], acc_sh.at[dst_v.at[q]],
                                 ssem[b], add=True)

                @pl.when(j + 2 < nch)
                def _():
                    pltpu.async_copy(dst_hbm.at[c, s, j + 2],
                                     dst_v.at[(q + 2) % RING_D],
                                     dsem[(q + 2) % RING_D])

                @pl.when(j + 1 < nch)
                def _():
                    @pl.when(j >= 1)
                    def _():
                        pltpu.make_async_copy(rows_v.at[1 - b],
                                              acc_sh.at[dst_v.at[q]],
                                              ssem[1 - b]).wait()
                    pltpu.async_copy(gtab_hbm.at[src_v.at[j + 1]],
                                     rows_v.at[1 - b], gsem[1 - b])
            return 0

        lax.fori_loop(0, nch // 4, step, 0)
        pltpu.make_async_copy(rows_v.at[(nch - 1) % NBUF],
                              acc_sh.at[dst_v.at[(nch - 1) % RING_D]],
                              ssem[(nch - 1) % NBUF]).wait()
        plsc.subcore_barrier()
        pltpu.sync_copy(acc_sh.at[pl.ds(base_row, ROWS_PER_TILE)],
                        out_hbm.at[c, pl.ds(base_row, ROWS_PER_TILE)])

    return agg


_agg128 = _make_agg(D_HID // 2, CHUNKS_PER_TILE)
_agg_l2 = _make_agg(D_OUT, CHUNKS_PER_TILE // 2)


def _dinv_block(deg_ref):
    d = deg_ref[0, 0, :] + deg_ref[0, 1, :] + 1.0
    return lax.rsqrt(d)[:, None]


def _tc0_body(x_ref, w0_ref, h_ref):
    h_ref[...] = jnp.dot(x_ref[...], w0_ref[...],
                         preferred_element_type=jnp.float32)


def _tc1_body(deg_ref, h_ref, g_ref):
    dinv = _dinv_block(deg_ref)
    g = h_ref[...] * dinv
    g_ref[0, :, :] = g[:, : D_HID // 2]
    g_ref[1, :, :] = g[:, D_HID // 2:]


def _tc2_body(deg_ref, s0_ref, g0_ref, b0_ref, w1_ref, g1_ref):
    dinv = _dinv_block(deg_ref)
    ya = (s0_ref[0, :, :] + g0_ref[0, :, :]) * dinv
    yb = (s0_ref[1, :, :] + g0_ref[1, :, :]) * dinv
    y = jnp.concatenate([ya, yb], axis=1) + b0_ref[...]
    out0 = jnp.maximum(y, 0.0)
    h1 = jnp.dot(out0, w1_ref[...], preferred_element_type=jnp.float32)
    g1_ref[...] = h1 * dinv


def _tc3_body(deg_ref, s1_ref, g1_ref, b1_ref, z_ref):
    dinv = _dinv_block(deg_ref)
    srow = s1_ref[0, :, :] + s1_ref[1, :, :] + g1_ref[...]
    z = srow * dinv + b1_ref[...]
    z_ref[...] = jnp.maximum(z, 0.0)


def _deg_spec():
    return pl.BlockSpec((1, 2, MBLK), lambda i: (i, 0, 0))


_tc0 = pl.pallas_call(
    _tc0_body,
    grid=(GRID_M,),
    in_specs=[
        pl.BlockSpec((MBLK, D_IN), lambda i: (i, 0)),
        pl.BlockSpec((D_IN, D_HID), lambda i: (0, 0)),
    ],
    out_specs=pl.BlockSpec((MBLK, D_HID), lambda i: (i, 0)),
    out_shape=jax.ShapeDtypeStruct((N, D_HID), jnp.float32),
)

_tc1 = pl.pallas_call(
    _tc1_body,
    grid=(GRID_M,),
    in_specs=[
        _deg_spec(),
        pl.BlockSpec((MBLK, D_HID), lambda i: (i, 0)),
    ],
    out_specs=pl.BlockSpec((2, MBLK, D_HID // 2), lambda i: (0, i, 0)),
    out_shape=jax.ShapeDtypeStruct((2, N, D_HID // 2), jnp.float32),
)

_tc2 = pl.pallas_call(
    _tc2_body,
    grid=(GRID_M,),
    in_specs=[
        _deg_spec(),
        pl.BlockSpec((2, MBLK, D_HID // 2), lambda i: (0, i, 0)),
        pl.BlockSpec((2, MBLK, D_HID // 2), lambda i: (0, i, 0)),
        pl.BlockSpec((1, D_HID), lambda i: (0, 0)),
        pl.BlockSpec((D_HID, D_OUT), lambda i: (0, 0)),
    ],
    out_specs=pl.BlockSpec((MBLK, D_OUT), lambda i: (i, 0)),
    out_shape=jax.ShapeDtypeStruct((N, D_OUT), jnp.float32),
)

_tc3 = pl.pallas_call(
    _tc3_body,
    grid=(GRID_M,),
    in_specs=[
        _deg_spec(),
        pl.BlockSpec((2, MBLK, D_OUT), lambda i: (0, i, 0)),
        pl.BlockSpec((MBLK, D_OUT), lambda i: (i, 0)),
        pl.BlockSpec((1, D_OUT), lambda i: (0, 0)),
    ],
    out_specs=pl.BlockSpec((MBLK, D_OUT), lambda i: (i, 0)),
    out_shape=jax.ShapeDtypeStruct((N, D_OUT), jnp.float32),
)


def kernel(x, edge_index, W0, b0, W1, b1):
    src = edge_index[0].astype(jnp.int32)
    dst = edge_index[1].astype(jnp.int32)
    npad = EPAD - E
    pad_i = jnp.arange(npad, dtype=jnp.int32)
    src_p = jnp.concatenate([src, pad_i % N])
    dst_p = jnp.concatenate([dst, N + pad_i % (NPAD - N)])
    src2 = jnp.stack([src_p, src_p + N]).reshape(2, NTILES, CHUNKS_PER_TILE, CH)
    src3 = src_p.reshape(NTILES, CHUNKS_PER_TILE, CH)
    dst3 = dst_p.reshape(NTILES, CHUNKS_PER_TILE, CH)
    dst2 = jnp.stack([dst3, dst3])
    half = CHUNKS_PER_TILE // 2
    src_l2 = jnp.stack([src3[:, :half], src3[:, half:]])
    dst_l2 = jnp.stack([dst3[:, :half], dst3[:, half:]])

    degp = _deg_kernel(dst_l2)
    degt = degp[:, :N, 0].reshape(2, GRID_M, MBLK).transpose(1, 0, 2)

    h0 = _tc0(x, W0)
    g0 = _tc1(degt, h0)
    s0 = _agg128(g0.reshape(2 * N, D_HID // 2), src2, dst2)

    g1 = _tc2(degt, s0[:, :N, :], g0, b0.reshape(1, D_HID), W1)
    s1 = _agg_l2(g1, src_l2, dst_l2)

    z = _tc3(degt, s1[:, :N, :], g1, b1.reshape(1, D_OUT))
    return z


# speedup vs baseline: 17.8115x; 1.1547x over previous
import functools

import jax
import jax.numpy as jnp
from jax import lax
from jax.experimental import pallas as pl
from jax.experimental.pallas import tpu as pltpu
from jax.experimental.pallas import tpu_sc as plsc

N = 10000
D_IN = 256
D_HID = 256
D_OUT = 128
E = 160000

NTILES = 16
CH = 128
NPAD = 10240
EPAD = 163840
ROWS_PER_TILE = NPAD // NTILES
CHUNKS_PER_TILE = EPAD // (NTILES * CH)
MBLK = 1000
GRID_M = N // MBLK

_MESH = plsc.VectorSubcoreMesh(core_axis_name="c", subcore_axis_name="s",
                               num_cores=2, num_subcores=NTILES)


def _fill(ref, nrows, ncols, value):
    npc = ncols // 16
    v = jnp.full((16,), value, jnp.float32)

    def body(i, _):
        for k in range(npc):
            ref[i, pl.ds(k * 16, 16)] = v
        return 0

    lax.fori_loop(0, nrows, body, 0)


_HCH = CHUNKS_PER_TILE // 2


@functools.partial(
    pl.kernel,
    out_type=jax.ShapeDtypeStruct((2, NPAD, 128), jnp.float32),
    mesh=_MESH,
    scratch_types=[
        pltpu.VMEM((CH,), jnp.int32),
        pltpu.VMEM((CH, 128), jnp.float32),
        pltpu.VMEM((CH, 128), jnp.float32),
        pltpu.VMEM_SHARED((NPAD, 128), jnp.float32),
    ],
)
def _deg_kernel(dst_hbm, out_hbm, dst_v, ones_v, zero_v, acc_sh):
    c = lax.axis_index("c")
    s = lax.axis_index("s")
    _fill(ones_v, CH, 128, 1.0)
    _fill(zero_v, CH, 128, 0.0)
    base_row = s * ROWS_PER_TILE
    for z in range(ROWS_PER_TILE // CH):
        pltpu.sync_copy(zero_v, acc_sh.at[pl.ds(base_row + z * CH, CH)])
    plsc.subcore_barrier()

    def body(j, _):
        pltpu.sync_copy(dst_hbm.at[c, s, j], dst_v)
        pltpu.sync_copy(ones_v, acc_sh.at[dst_v], add=True)
        return 0

    lax.fori_loop(0, _HCH, body, 0)
    plsc.subcore_barrier()
    pltpu.sync_copy(acc_sh.at[pl.ds(base_row, ROWS_PER_TILE)],
                    out_hbm.at[c, pl.ds(base_row, ROWS_PER_TILE)])


NBUF = 2


def _make_agg(dh, nch):
    @functools.partial(
        pl.kernel,
        out_type=jax.ShapeDtypeStruct((2, NPAD, dh), jnp.float32),
        mesh=_MESH,
        scratch_types=[
            pltpu.VMEM((nch, CH), jnp.int32),
            pltpu.VMEM((NBUF, CH), jnp.int32),
            pltpu.VMEM((NBUF, CH, dh), jnp.float32),
            pltpu.VMEM_SHARED((NPAD, dh), jnp.float32),
        ] + [pltpu.SemaphoreType.DMA] * (3 * NBUF),
    )
    def agg(gtab_hbm, src_hbm, dst_hbm, out_hbm, src_v, dst_v, rows_v,
            acc_sh, *sems):
        gsem = sems[:NBUF]
        ssem = sems[NBUF:2 * NBUF]
        dsem = sems[2 * NBUF:]
        c = lax.axis_index("c")
        s = lax.axis_index("s")
        _fill(rows_v.at[0], CH, dh, 0.0)
        base_row = s * ROWS_PER_TILE
        for z in range(ROWS_PER_TILE // CH):
            pltpu.sync_copy(rows_v.at[0],
                            acc_sh.at[pl.ds(base_row + z * CH, CH)])
        pltpu.sync_copy(src_hbm.at[c, s], src_v)

        for b in range(NBUF):
            pltpu.async_copy(dst_hbm.at[c, s, b], dst_v.at[b], dsem[b])
            pltpu.async_copy(gtab_hbm.at[src_v.at[b]], rows_v.at[b], gsem[b])
        plsc.subcore_barrier()

        def step(t, _):
            for b in range(NBUF):
                j = t * NBUF + b
                pltpu.make_async_copy(gtab_hbm.at[src_v.at[j]], rows_v.at[b],
                                      gsem[b]).wait()
                pltpu.make_async_copy(dst_hbm.at[c, s, j], dst_v.at[b],
                                      dsem[b]).wait()
                pltpu.async_copy(rows_v.at[b], acc_sh.at[dst_v.at[b]],
                                 ssem[b], add=True)

                @pl.when(t < nch // NBUF - 1)
                def _():
                    pltpu.make_async_copy(rows_v.at[b],
                                          acc_sh.at[dst_v.at[b]],
                                          ssem[b]).wait()
                    pltpu.async_copy(dst_hbm.at[c, s, j + NBUF],
                                     dst_v.at[b], dsem[b])
                    pltpu.async_copy(gtab_hbm.at[src_v.at[j + NBUF]],
                                     rows_v.at[b], gsem[b])
            return 0

        lax.fori_loop(0, nch // NBUF, step, 0)
        for b in range(NBUF):
            pltpu.make_async_copy(rows_v.at[b], acc_sh.at[dst_v.at[b]],
                                  ssem[b]).wait()
        plsc.subcore_barrier()
        pltpu.sync_copy(acc_sh.at[pl.ds(base_row, ROWS_PER_TILE)],
                        out_hbm.at[c, pl.ds(base_row, ROWS_PER_TILE)])

    return agg


_agg128 = _make_agg(D_HID // 2, CHUNKS_PER_TILE)
_agg_l2 = _make_agg(D_OUT, CHUNKS_PER_TILE // 2)


def _dinv_block(deg_ref):
    d = deg_ref[0, 0, :] + deg_ref[0, 1, :] + 1.0
    return lax.rsqrt(d)[:, None]


def _tc0_body(x_ref, w0_ref, h_ref):
    h_ref[...] = jnp.dot(x_ref[...], w0_ref[...],
                         preferred_element_type=jnp.float32)


def _tc1_body(deg_ref, h_ref, g_ref):
    dinv = _dinv_block(deg_ref)
    g = h_ref[...] * dinv
    g_ref[0, :, :] = g[:, : D_HID // 2]
    g_ref[1, :, :] = g[:, D_HID // 2:]


def _tc2_body(deg_ref, s0_ref, g0_ref, b0_ref, w1_ref, g1_ref):
    dinv = _dinv_block(deg_ref)
    ya = (s0_ref[0, :, :] + g0_ref[0, :, :]) * dinv
    yb = (s0_ref[1, :, :] + g0_ref[1, :, :]) * dinv
    y = jnp.concatenate([ya, yb], axis=1) + b0_ref[...]
    out0 = jnp.maximum(y, 0.0)
    h1 = jnp.dot(out0, w1_ref[...], preferred_element_type=jnp.float32)
    g1_ref[...] = h1 * dinv


def _tc3_body(deg_ref, s1_ref, g1_ref, b1_ref, z_ref):
    dinv = _dinv_block(deg_ref)
    srow = s1_ref[0, :, :] + s1_ref[1, :, :] + g1_ref[...]
    z = srow * dinv + b1_ref[...]
    z_ref[...] = jnp.maximum(z, 0.0)


def _deg_spec():
    return pl.BlockSpec((1, 2, MBLK), lambda i: (i, 0, 0))


_tc0 = pl.pallas_call(
    _tc0_body,
    grid=(GRID_M,),
    in_specs=[
        pl.BlockSpec((MBLK, D_IN), lambda i: (i, 0)),
        pl.BlockSpec((D_IN, D_HID), lambda i: (0, 0)),
    ],
    out_specs=pl.BlockSpec((MBLK, D_HID), lambda i: (i, 0)),
    out_shape=jax.ShapeDtypeStruct((N, D_HID), jnp.float32),
)

_tc1 = pl.pallas_call(
    _tc1_body,
    grid=(GRID_M,),
    in_specs=[
        _deg_spec(),
        pl.BlockSpec((MBLK, D_HID), lambda i: (i, 0)),
    ],
    out_specs=pl.BlockSpec((2, MBLK, D_HID // 2), lambda i: (0, i, 0)),
    out_shape=jax.ShapeDtypeStruct((2, N, D_HID // 2), jnp.float32),
)

_tc2 = pl.pallas_call(
    _tc2_body,
    grid=(GRID_M,),
    in_specs=[
        _deg_spec(),
        pl.BlockSpec((2, MBLK, D_HID // 2), lambda i: (0, i, 0)),
        pl.BlockSpec((2, MBLK, D_HID // 2), lambda i: (0, i, 0)),
        pl.BlockSpec((1, D_HID), lambda i: (0, 0)),
        pl.BlockSpec((D_HID, D_OUT), lambda i: (0, 0)),
    ],
    out_specs=pl.BlockSpec((MBLK, D_OUT), lambda i: (i, 0)),
    out_shape=jax.ShapeDtypeStruct((N, D_OUT), jnp.float32),
)

_tc3 = pl.pallas_call(
    _tc3_body,
    grid=(GRID_M,),
    in_specs=[
        _deg_spec(),
        pl.BlockSpec((2, MBLK, D_OUT), lambda i: (0, i, 0)),
        pl.BlockSpec((MBLK, D_OUT), lambda i: (i, 0)),
        pl.BlockSpec((1, D_OUT), lambda i: (0, 0)),
    ],
    out_specs=pl.BlockSpec((MBLK, D_OUT), lambda i: (i, 0)),
    out_shape=jax.ShapeDtypeStruct((N, D_OUT), jnp.float32),
)


def kernel(x, edge_index, W0, b0, W1, b1):
    src = edge_index[0].astype(jnp.int32)
    dst = edge_index[1].astype(jnp.int32)
    npad = EPAD - E
    pad_i = jnp.arange(npad, dtype=jnp.int32)
    src_p = jnp.concatenate([src, pad_i % N])
    dst_p = jnp.concatenate([dst, N + pad_i % (NPAD - N)])
    src2 = jnp.stack([src_p, src_p + N]).reshape(2, NTILES, CHUNKS_PER_TILE, CH)
    src3 = src_p.reshape(NTILES, CHUNKS_PER_TILE, CH)
    dst3 = dst_p.reshape(NTILES, CHUNKS_PER_TILE, CH)
    dst2 = jnp.stack([dst3, dst3])
    half = CHUNKS_PER_TILE // 2
    src_l2 = jnp.stack([src3[:, :half], src3[:, half:]])
    dst_l2 = jnp.stack([dst3[:, :half], dst3[:, half:]])

    degp = _deg_kernel(dst_l2)
    degt = degp[:, :N, 0].reshape(2, GRID_M, MBLK).transpose(1, 0, 2)

    h0 = _tc0(x, W0)
    g0 = _tc1(degt, h0)
    s0 = _agg128(g0.reshape(2 * N, D_HID // 2), src2, dst2)

    g1 = _tc2(degt, s0[:, :N, :], g0, b0.reshape(1, D_HID), W1)
    s1 = _agg_l2(g1, src_l2, dst_l2)

    z = _tc3(degt, s1[:, :N, :], g1, b1.reshape(1, D_OUT))
    return z


# speedup vs baseline: 18.9063x; 1.0615x over previous
import functools

import jax
import jax.numpy as jnp
from jax import lax
from jax.experimental import pallas as pl
from jax.experimental.pallas import tpu as pltpu
from jax.experimental.pallas import tpu_sc as plsc

N = 10000
D_IN = 256
D_HID = 256
D_OUT = 128
E = 160000

NTILES = 16
CH = 128
NPAD = 10240
EPAD = 163840
ROWS_PER_TILE = NPAD // NTILES
CHUNKS_PER_TILE = EPAD // (NTILES * CH)
MBLK = 1000
GRID_M = N // MBLK

_MESH = plsc.VectorSubcoreMesh(core_axis_name="c", subcore_axis_name="s",
                               num_cores=2, num_subcores=NTILES)


def _fill(ref, nrows, ncols, value):
    npc = ncols // 16
    v = jnp.full((16,), value, jnp.float32)

    def body(i, _):
        for k in range(npc):
            ref[i, pl.ds(k * 16, 16)] = v
        return 0

    lax.fori_loop(0, nrows, body, 0)


_HCH = CHUNKS_PER_TILE // 2


@functools.partial(
    pl.kernel,
    out_type=jax.ShapeDtypeStruct((2, NPAD, 128), jnp.float32),
    mesh=_MESH,
    scratch_types=[
        pltpu.VMEM((4, CH), jnp.int32),
        pltpu.VMEM((CH, 128), jnp.float32),
        pltpu.VMEM((CH, 128), jnp.float32),
        pltpu.VMEM_SHARED((NPAD, 128), jnp.float32),
    ] + [pltpu.SemaphoreType.DMA] * 8,
)
def _deg_kernel(dst_hbm, out_hbm, dst_v, ones_v, zero_v, acc_sh, *sems):
    dsem = sems[:4]
    ssem = sems[4:]
    c = lax.axis_index("c")
    s = lax.axis_index("s")
    _fill(ones_v, CH, 128, 1.0)
    _fill(zero_v, CH, 128, 0.0)
    base_row = s * ROWS_PER_TILE
    for z in range(ROWS_PER_TILE // CH):
        pltpu.sync_copy(zero_v, acc_sh.at[pl.ds(base_row + z * CH, CH)])
    plsc.subcore_barrier()

    for q in range(2):
        pltpu.async_copy(dst_hbm.at[c, s, q], dst_v.at[q], dsem[q])

    def body(t, _):
        for u in range(4):
            j = t * 4 + u
            pltpu.make_async_copy(dst_hbm.at[c, s, j], dst_v.at[u],
                                  dsem[u]).wait()
            pltpu.async_copy(ones_v, acc_sh.at[dst_v.at[u]], ssem[u],
                             add=True)

            @pl.when(j + 2 < _HCH)
            def _():
                @pl.when(j >= 2)
                def _():
                    pltpu.make_async_copy(ones_v,
                                          acc_sh.at[dst_v.at[(u + 2) % 4]],
                                          ssem[(u + 2) % 4]).wait()
                pltpu.async_copy(dst_hbm.at[c, s, j + 2],
                                 dst_v.at[(u + 2) % 4], dsem[(u + 2) % 4])
        return 0

    lax.fori_loop(0, _HCH // 4, body, 0)
    for q in range(4):
        pltpu.make_async_copy(ones_v, acc_sh.at[dst_v.at[q]],
                              ssem[q]).wait()
    plsc.subcore_barrier()
    pltpu.sync_copy(acc_sh.at[pl.ds(base_row, ROWS_PER_TILE)],
                    out_hbm.at[c, pl.ds(base_row, ROWS_PER_TILE)])


NBUF = 2


def _make_agg(dh, nch):
    @functools.partial(
        pl.kernel,
        out_type=jax.ShapeDtypeStruct((2, NPAD, dh), jnp.float32),
        mesh=_MESH,
        scratch_types=[
            pltpu.VMEM((nch, CH), jnp.int32),
            pltpu.VMEM((NBUF, CH), jnp.int32),
            pltpu.VMEM((NBUF, CH, dh), jnp.float32),
            pltpu.VMEM_SHARED((NPAD, dh), jnp.float32),
        ] + [pltpu.SemaphoreType.DMA] * (3 * NBUF),
    )
    def agg(gtab_hbm, src_hbm, dst_hbm, out_hbm, src_v, dst_v, rows_v,
            acc_sh, *sems):
        gsem = sems[:NBUF]
        ssem = sems[NBUF:2 * NBUF]
        dsem = sems[2 * NBUF:]
        c = lax.axis_index("c")
        s = lax.axis_index("s")
        _fill(rows_v.at[0], CH, dh, 0.0)
        base_row = s * ROWS_PER_TILE
        for z in range(ROWS_PER_TILE // CH):
            pltpu.sync_copy(rows_v.at[0],
                            acc_sh.at[pl.ds(base_row + z * CH, CH)])
        pltpu.sync_copy(src_hbm.at[c, s], src_v)

        for b in range(NBUF):
            pltpu.async_copy(dst_hbm.at[c, s, b], dst_v.at[b], dsem[b])
            pltpu.async_copy(gtab_hbm.at[src_v.at[b]], rows_v.at[b], gsem[b])
        plsc.subcore_barrier()

        def step(t, _):
            for b in range(NBUF):
                j = t * NBUF + b
                pltpu.make_async_copy(gtab_hbm.at[src_v.at[j]], rows_v.at[b],
                                      gsem[b]).wait()
                pltpu.make_async_copy(dst_hbm.at[c, s, j], dst_v.at[b],
                                      dsem[b]).wait()
                pltpu.async_copy(rows_v.at[b], acc_sh.at[dst_v.at[b]],
                                 ssem[b], add=True)

                @pl.when(t < nch // NBUF - 1)
                def _():
                    pltpu.make_async_copy(rows_v.at[b],
                                          acc_sh.at[dst_v.at[b]],
                                          ssem[b]).wait()
                    pltpu.async_copy(dst_hbm.at[c, s, j + NBUF],
                                     dst_v.at[b], dsem[b])
                    pltpu.async_copy(gtab_hbm.at[src_v.at[j + NBUF]],
                                     rows_v.at[b], gsem[b])
            return 0

        lax.fori_loop(0, nch // NBUF, step, 0)
        for b in range(NBUF):
            pltpu.make_async_copy(rows_v.at[b], acc_sh.at[dst_v.at[b]],
                                  ssem[b]).wait()
        plsc.subcore_barrier()
        pltpu.sync_copy(acc_sh.at[pl.ds(base_row, ROWS_PER_TILE)],
                        out_hbm.at[c, pl.ds(base_row, ROWS_PER_TILE)])

    return agg


_agg128 = _make_agg(D_HID // 2, CHUNKS_PER_TILE)
_agg_l2 = _make_agg(D_OUT, CHUNKS_PER_TILE // 2)


def _dinv_block(deg_ref):
    d = deg_ref[0, 0, :] + deg_ref[0, 1, :] + 1.0
    return lax.rsqrt(d)[:, None]


def _tc0_body(x_ref, w0_ref, h_ref):
    h_ref[...] = jnp.dot(x_ref[...], w0_ref[...],
                         preferred_element_type=jnp.float32)


def _tc1_body(deg_ref, h_ref, g_ref):
    dinv = _dinv_block(deg_ref)
    g = h_ref[...] * dinv
    g_ref[0, :, :] = g[:, : D_HID // 2]
    g_ref[1, :, :] = g[:, D_HID // 2:]


def _tc2_body(deg_ref, s0_ref, g0_ref, b0_ref, w1_ref, g1_ref):
    dinv = _dinv_block(deg_ref)
    ya = (s0_ref[0, :, :] + g0_ref[0, :, :]) * dinv
    yb = (s0_ref[1, :, :] + g0_ref[1, :, :]) * dinv
    y = jnp.concatenate([ya, yb], axis=1) + b0_ref[...]
    out0 = jnp.maximum(y, 0.0)
    h1 = jnp.dot(out0, w1_ref[...], preferred_element_type=jnp.float32)
    g1_ref[...] = h1 * dinv


def _tc3_body(deg_ref, s1_ref, g1_ref, b1_ref, z_ref):
    dinv = _dinv_block(deg_ref)
    srow = s1_ref[0, :, :] + s1_ref[1, :, :] + g1_ref[...]
    z = srow * dinv + b1_ref[...]
    z_ref[...] = jnp.maximum(z, 0.0)


def _deg_spec():
    return pl.BlockSpec((1, 2, MBLK), lambda i: (i, 0, 0))


_tc0 = pl.pallas_call(
    _tc0_body,
    grid=(GRID_M,),
    in_specs=[
        pl.BlockSpec((MBLK, D_IN), lambda i: (i, 0)),
        pl.BlockSpec((D_IN, D_HID), lambda i: (0, 0)),
    ],
    out_specs=pl.BlockSpec((MBLK, D_HID), lambda i: (i, 0)),
    out_shape=jax.ShapeDtypeStruct((N, D_HID), jnp.float32),
)

_tc1 = pl.pallas_call(
    _tc1_body,
    grid=(GRID_M,),
    in_specs=[
        _deg_spec(),
        pl.BlockSpec((MBLK, D_HID), lambda i: (i, 0)),
    ],
    out_specs=pl.BlockSpec((2, MBLK, D_HID // 2), lambda i: (0, i, 0)),
    out_shape=jax.ShapeDtypeStruct((2, N, D_HID // 2), jnp.float32),
)

_tc2 = pl.pallas_call(
    _tc2_body,
    grid=(GRID_M,),
    in_specs=[
        _deg_spec(),
        pl.BlockSpec((2, MBLK, D_HID // 2), lambda i: (0, i, 0)),
        pl.BlockSpec((2, MBLK, D_HID // 2), lambda i: (0, i, 0)),
        pl.BlockSpec((1, D_HID), lambda i: (0, 0)),
        pl.BlockSpec((D_HID, D_OUT), lambda i: (0, 0)),
    ],
    out_specs=pl.BlockSpec((MBLK, D_OUT), lambda i: (i, 0)),
    out_shape=jax.ShapeDtypeStruct((N, D_OUT), jnp.float32),
)

_tc3 = pl.pallas_call(
    _tc3_body,
    grid=(GRID_M,),
    in_specs=[
        _deg_spec(),
        pl.BlockSpec((2, MBLK, D_OUT), lambda i: (0, i, 0)),
        pl.BlockSpec((MBLK, D_OUT), lambda i: (i, 0)),
        pl.BlockSpec((1, D_OUT), lambda i: (0, 0)),
    ],
    out_specs=pl.BlockSpec((MBLK, D_OUT), lambda i: (i, 0)),
    out_shape=jax.ShapeDtypeStruct((N, D_OUT), jnp.float32),
)


def kernel(x, edge_index, W0, b0, W1, b1):
    src = edge_index[0].astype(jnp.int32)
    dst = edge_index[1].astype(jnp.int32)
    npad = EPAD - E
    pad_i = jnp.arange(npad, dtype=jnp.int32)
    src_p = jnp.concatenate([src, pad_i % N])
    dst_p = jnp.concatenate([dst, N + pad_i % (NPAD - N)])
    src2 = jnp.stack([src_p, src_p + N]).reshape(2, NTILES, CHUNKS_PER_TILE, CH)
    src3 = src_p.reshape(NTILES, CHUNKS_PER_TILE, CH)
    dst3 = dst_p.reshape(NTILES, CHUNKS_PER_TILE, CH)
    dst2 = jnp.stack([dst3, dst3])
    half = CHUNKS_PER_TILE // 2
    src_l2 = jnp.stack([src3[:, :half], src3[:, half:]])
    dst_l2 = jnp.stack([dst3[:, :half], dst3[:, half:]])

    degp = _deg_kernel(dst_l2)
    degt = degp[:, :N, 0].reshape(2, GRID_M, MBLK).transpose(1, 0, 2)

    h0 = _tc0(x, W0)
    g0 = _tc1(degt, h0)
    s0 = _agg128(g0.reshape(2 * N, D_HID // 2), src2, dst2)

    g1 = _tc2(degt, s0[:, :N, :], g0, b0.reshape(1, D_HID), W1)
    s1 = _agg_l2(g1, src_l2, dst_l2)

    z = _tc3(degt, s1[:, :N, :], g1, b1.reshape(1, D_OUT))
    return z


# speedup vs baseline: 19.1439x; 1.0126x over previous
import functools

import jax
import jax.numpy as jnp
from jax import lax
from jax.experimental import pallas as pl
from jax.experimental.pallas import tpu as pltpu
from jax.experimental.pallas import tpu_sc as plsc

N = 10000
D_IN = 256
D_HID = 256
D_OUT = 128
E = 160000

NTILES = 16
CH = 128
NPAD = 10240
EPAD = 163840
ROWS_PER_TILE = NPAD // NTILES
CHUNKS_PER_TILE = EPAD // (NTILES * CH)
MBLK = 1000
GRID_M = N // MBLK

_MESH = plsc.VectorSubcoreMesh(core_axis_name="c", subcore_axis_name="s",
                               num_cores=2, num_subcores=NTILES)


def _fill(ref, nrows, ncols, value):
    npc = ncols // 16
    v = jnp.full((16,), value, jnp.float32)

    def body(i, _):
        for k in range(npc):
            ref[i, pl.ds(k * 16, 16)] = v
        return 0

    lax.fori_loop(0, nrows, body, 0)


_HCH = CHUNKS_PER_TILE // 2


@functools.partial(
    pl.kernel,
    out_type=jax.ShapeDtypeStruct((2, NPAD, 128), jnp.float32),
    mesh=_MESH,
    scratch_types=[
        pltpu.VMEM((4, CH), jnp.int32),
        pltpu.VMEM((CH, 128), jnp.float32),
        pltpu.VMEM((CH, 128), jnp.float32),
        pltpu.VMEM_SHARED((NPAD, 128), jnp.float32),
    ] + [pltpu.SemaphoreType.DMA] * 9,
)
def _deg_kernel(dst_hbm, out_hbm, dst_v, ones_v, zero_v, acc_sh, *sems):
    dsem = sems[:4]
    ssem = sems[4:8]
    zsem = sems[8]
    c = lax.axis_index("c")
    s = lax.axis_index("s")
    _fill(ones_v, CH, 128, 1.0)
    _fill(zero_v, CH, 128, 0.0)
    base_row = s * ROWS_PER_TILE
    for z in range(ROWS_PER_TILE // CH):
        pltpu.async_copy(zero_v, acc_sh.at[pl.ds(base_row + z * CH, CH)],
                         zsem)
    for q in range(2):
        pltpu.async_copy(dst_hbm.at[c, s, q], dst_v.at[q], dsem[q])
    for z in range(ROWS_PER_TILE // CH):
        pltpu.make_async_copy(zero_v, acc_sh.at[pl.ds(base_row + z * CH, CH)],
                              zsem).wait()
    plsc.subcore_barrier()

    def body(t, _):
        for u in range(4):
            j = t * 4 + u
            pltpu.make_async_copy(dst_hbm.at[c, s, j], dst_v.at[u],
                                  dsem[u]).wait()
            pltpu.async_copy(ones_v, acc_sh.at[dst_v.at[u]], ssem[u],
                             add=True)

            @pl.when(j + 2 < _HCH)
            def _():
                @pl.when(j >= 2)
                def _():
                    pltpu.make_async_copy(ones_v,
                                          acc_sh.at[dst_v.at[(u + 2) % 4]],
                                          ssem[(u + 2) % 4]).wait()
                pltpu.async_copy(dst_hbm.at[c, s, j + 2],
                                 dst_v.at[(u + 2) % 4], dsem[(u + 2) % 4])
        return 0

    lax.fori_loop(0, _HCH // 4, body, 0)
    for q in range(4):
        pltpu.make_async_copy(ones_v, acc_sh.at[dst_v.at[q]],
                              ssem[q]).wait()
    plsc.subcore_barrier()
    pltpu.sync_copy(acc_sh.at[pl.ds(base_row, ROWS_PER_TILE)],
                    out_hbm.at[c, pl.ds(base_row, ROWS_PER_TILE)])


NBUF = 2


def _make_agg(dh, nch):
    @functools.partial(
        pl.kernel,
        out_type=jax.ShapeDtypeStruct((2, NPAD, dh), jnp.float32),
        mesh=_MESH,
        scratch_types=[
            pltpu.VMEM((nch, CH), jnp.int32),
            pltpu.VMEM((NBUF, CH), jnp.int32),
            pltpu.VMEM((NBUF, CH, dh), jnp.float32),
            pltpu.VMEM_SHARED((NPAD, dh), jnp.float32),
        ] + [pltpu.SemaphoreType.DMA] * (3 * NBUF + 1),
    )
    def agg(gtab_hbm, src_hbm, dst_hbm, out_hbm, src_v, dst_v, rows_v,
            acc_sh, *sems):
        gsem = sems[:NBUF]
        ssem = sems[NBUF:2 * NBUF]
        dsem = sems[2 * NBUF:3 * NBUF]
        zsem = sems[3 * NBUF]
        c = lax.axis_index("c")
        s = lax.axis_index("s")
        _fill(rows_v.at[0], CH, dh, 0.0)
        base_row = s * ROWS_PER_TILE
        for z in range(ROWS_PER_TILE // CH):
            pltpu.async_copy(rows_v.at[0],
                            acc_sh.at[pl.ds(base_row + z * CH, CH)], zsem)
        pltpu.sync_copy(src_hbm.at[c, s], src_v)
        for z in range(ROWS_PER_TILE // CH):
            pltpu.make_async_copy(
                rows_v.at[0], acc_sh.at[pl.ds(base_row + z * CH, CH)],
                zsem).wait()

        for b in range(NBUF):
            pltpu.async_copy(dst_hbm.at[c, s, b], dst_v.at[b], dsem[b])
            pltpu.async_copy(gtab_hbm.at[src_v.at[b]], rows_v.at[b], gsem[b])
        plsc.subcore_barrier()

        def step(t, _):
            for b in range(NBUF):
                j = t * NBUF + b
                pltpu.make_async_copy(gtab_hbm.at[src_v.at[j]], rows_v.at[b],
                                      gsem[b]).wait()
                pltpu.make_async_copy(dst_hbm.at[c, s, j], dst_v.at[b],
                                      dsem[b]).wait()
                pltpu.async_copy(rows_v.at[b], acc_sh.at[dst_v.at[b]],
                                 ssem[b], add=True)

                @pl.when(t < nch // NBUF - 1)
                def _():
                    pltpu.make_async_copy(rows_v.at[b],
                                          acc_sh.at[dst_v.at[b]],
                                          ssem[b]).wait()
                    pltpu.async_copy(dst_hbm.at[c, s, j + NBUF],
                                     dst_v.at[b], dsem[b])
                    pltpu.async_copy(gtab_hbm.at[src_v.at[j + NBUF]],
                                     rows_v.at[b], gsem[b])
            return 0

        lax.fori_loop(0, nch // NBUF, step, 0)
        for b in range(NBUF):
            pltpu.make_async_copy(rows_v.at[b], acc_sh.at[dst_v.at[b]],
                                  ssem[b]).wait()
        plsc.subcore_barrier()
        pltpu.sync_copy(acc_sh.at[pl.ds(base_row, ROWS_PER_TILE)],
                        out_hbm.at[c, pl.ds(base_row, ROWS_PER_TILE)])

    return agg


_agg128 = _make_agg(D_HID // 2, CHUNKS_PER_TILE)
_agg_l2 = _make_agg(D_OUT, CHUNKS_PER_TILE // 2)


def _dinv_block(deg_ref):
    d = deg_ref[0, 0, :] + deg_ref[0, 1, :] + 1.0
    return lax.rsqrt(d)[:, None]


def _tc0_body(x_ref, w0_ref, h_ref):
    h_ref[...] = jnp.dot(x_ref[...], w0_ref[...],
                         preferred_element_type=jnp.float32)


def _tc1_body(deg_ref, h_ref, g_ref):
    dinv = _dinv_block(deg_ref)
    g = h_ref[...] * dinv
    g_ref[0, :, :] = g[:, : D_HID // 2]
    g_ref[1, :, :] = g[:, D_HID // 2:]


def _tc2_body(deg_ref, s0_ref, g0_ref, b0_ref, w1_ref, g1_ref):
    dinv = _dinv_block(deg_ref)
    ya = (s0_ref[0, :, :] + g0_ref[0, :, :]) * dinv
    yb = (s0_ref[1, :, :] + g0_ref[1, :, :]) * dinv
    y = jnp.concatenate([ya, yb], axis=1) + b0_ref[...]
    out0 = jnp.maximum(y, 0.0)
    h1 = jnp.dot(out0, w1_ref[...], preferred_element_type=jnp.float32)
    g1_ref[...] = h1 * dinv


def _tc3_body(deg_ref, s1_ref, g1_ref, b1_ref, z_ref):
    dinv = _dinv_block(deg_ref)
    srow = s1_ref[0, :, :] + s1_ref[1, :, :] + g1_ref[...]
    z = srow * dinv + b1_ref[...]
    z_ref[...] = jnp.maximum(z, 0.0)


def _deg_spec():
    return pl.BlockSpec((1, 2, MBLK), lambda i: (i, 0, 0))


_tc0 = pl.pallas_call(
    _tc0_body,
    grid=(GRID_M,),
    in_specs=[
        pl.BlockSpec((MBLK, D_IN), lambda i: (i, 0)),
        pl.BlockSpec((D_IN, D_HID), lambda i: (0, 0)),
    ],
    out_specs=pl.BlockSpec((MBLK, D_HID), lambda i: (i, 0)),
    out_shape=jax.ShapeDtypeStruct((N, D_HID), jnp.float32),
)

_tc1 = pl.pallas_call(
    _tc1_body,
    grid=(GRID_M,),
    in_specs=[
        _deg_spec(),
        pl.BlockSpec((MBLK, D_HID), lambda i: (i, 0)),
    ],
    out_specs=pl.BlockSpec((2, MBLK, D_HID // 2), lambda i: (0, i, 0)),
    out_shape=jax.ShapeDtypeStruct((2, N, D_HID // 2), jnp.float32),
)

_tc2 = pl.pallas_call(
    _tc2_body,
    grid=(GRID_M,),
    in_specs=[
        _deg_spec(),
        pl.BlockSpec((2, MBLK, D_HID // 2), lambda i: (0, i, 0)),
        pl.BlockSpec((2, MBLK, D_HID // 2), lambda i: (0, i, 0)),
        pl.BlockSpec((1, D_HID), lambda i: (0, 0)),
        pl.BlockSpec((D_HID, D_OUT), lambda i: (0, 0)),
    ],
    out_specs=pl.BlockSpec((MBLK, D_OUT), lambda i: (i, 0)),
    out_shape=jax.ShapeDtypeStruct((N, D_OUT), jnp.float32),
)

_tc3 = pl.pallas_call(
    _tc3_body,
    grid=(GRID_M,),
    in_specs=[
        _deg_spec(),
        pl.BlockSpec((2, MBLK, D_OUT), lambda i: (0, i, 0)),
        pl.BlockSpec((MBLK, D_OUT), lambda i: (i, 0)),
        pl.BlockSpec((1, D_OUT), lambda i: (0, 0)),
    ],
    out_specs=pl.BlockSpec((MBLK, D_OUT), lambda i: (i, 0)),
    out_shape=jax.ShapeDtypeStruct((N, D_OUT), jnp.float32),
)


def kernel(x, edge_index, W0, b0, W1, b1):
    src = edge_index[0].astype(jnp.int32)
    dst = edge_index[1].astype(jnp.int32)
    npad = EPAD - E
    pad_i = jnp.arange(npad, dtype=jnp.int32)
    src_p = jnp.concatenate([src, pad_i % N])
    dst_p = jnp.concatenate([dst, N + pad_i % (NPAD - N)])
    src2 = jnp.stack([src_p, src_p + N]).reshape(2, NTILES, CHUNKS_PER_TILE, CH)
    src3 = src_p.reshape(NTILES, CHUNKS_PER_TILE, CH)
    dst3 = dst_p.reshape(NTILES, CHUNKS_PER_TILE, CH)
    dst2 = jnp.stack([dst3, dst3])
    half = CHUNKS_PER_TILE // 2
    src_l2 = jnp.stack([src3[:, :half], src3[:, half:]])
    dst_l2 = jnp.stack([dst3[:, :half], dst3[:, half:]])

    degp = _deg_kernel(dst_l2)
    degt = degp[:, :N, 0].reshape(2, GRID_M, MBLK).transpose(1, 0, 2)

    h0 = _tc0(x, W0)
    g0 = _tc1(degt, h0)
    s0 = _agg128(g0.reshape(2 * N, D_HID // 2), src2, dst2)

    g1 = _tc2(degt, s0[:, :N, :], g0, b0.reshape(1, D_HID), W1)
    s1 = _agg_l2(g1, src_l2, dst_l2)

    z = _tc3(degt, s1[:, :N, :], g1, b1.reshape(1, D_OUT))
    return z
